# baseline (speedup 1.0000x reference)
import jax
import jax.numpy as jnp
from jax import lax
from jax.experimental import pallas as pl
from jax.experimental.pallas import tpu as pltpu
from jax.experimental.pallas import tpu_sc as plsc

N = 320000
NT = 17
R = 8
D = 128
L = 16
NC, NS = 2, 16
NW = NC * NS
ROWS_PER_W = N // NW
C = 80
NCHUNK = ROWS_PER_W // C

NPAIR = 8
W32 = D // 2
TBL_ROWS = NPAIR * 64 + R


def _sc_body(w_hbm, x_hbm, out_hbm, wbuf, tbl, xbuf, outbuf):
    wid = lax.axis_index("s") * NC + lax.axis_index("c")
    base = wid * ROWS_PER_W

    pltpu.sync_copy(w_hbm, wbuf)

    iota = lax.iota(jnp.int32, L)

    def to_bf16_bits(v):
        w = lax.bitcast_convert_type(v, jnp.int32)
        rounded = w + 0x7FFF + ((w >> 16) & 1)
        return (rounded >> 16) & 0xFFFF

    ebase = [(iota + k2 * L) * TBL_ROWS for k2 in range(4)]

    def pack_row_to(vs, r):
        for k2 in range(4):
            lo = to_bf16_bits(vs[2 * k2])
            hi = to_bf16_bits(vs[2 * k2 + 1])
            word = lax.bitcast_convert_type(lo | (hi << 16), jnp.float32)
            plsc.store_scatter(tbl, [ebase[k2] + r], word)

    def build_pair(p, _):
        def build_ab(ab, _):
            a = ab // R
            b = ab - a * R
            src_a = ((2 * p) * R + a) * D
            src_b = ((2 * p + 1) * R + b) * D
            vs = [wbuf[pl.ds(src_a + k * L, L)] + wbuf[pl.ds(src_b + k * L, L)]
                  for k in range(8)]
            pack_row_to(vs, p * 64 + ab)
            return 0
        lax.fori_loop(0, 64, build_ab, 0)
        return 0
    lax.fori_loop(0, NPAIR, build_pair, 0)

    def build_single(r, _):
        src = (16 * R + r) * D
        vs = [wbuf[pl.ds(src + k * L, L)] for k in range(8)]
        pack_row_to(vs, NPAIR * 64 + r)
        return 0
    lax.fori_loop(0, R, build_single, 0)

    xcol_base = iota * NT
    rowb_base = iota * D

    def chunk_body(j, _):
        row0 = base + j * C
        pltpu.sync_copy(x_hbm.at[pl.ds(row0 * NT, C * NT)], xbuf)

        def group_body(g, _):
            r0 = g * L
            xidx = xcol_base + r0 * NT
            cols = [plsc.load_gather(xbuf, [xidx + i]) for i in range(NT)]
            offs = []
            for p in range(NPAIR):
                offs.append(cols[2 * p] * R + cols[2 * p + 1] + p * 64)
            offs.append(cols[16] + NPAIR * 64)
            rowb = rowb_base + r0 * D
            for e in range(W32):
                acc_lo = None
                acc_hi = None
                for t in range(NPAIR + 1):
                    g = lax.bitcast_convert_type(plsc.load_gather(
                        tbl.at[pl.ds(e * TBL_ROWS, (W32 - e) * TBL_ROWS)],
                        [offs[t]]), jnp.int32)
                    lo = lax.bitcast_convert_type(g << 16, jnp.float32)
                    hi = lax.bitcast_convert_type(g & jnp.int32(-65536),
                                                  jnp.float32)
                    acc_lo = lo if acc_lo is None else acc_lo + lo
                    acc_hi = hi if acc_hi is None else acc_hi + hi
                k2 = e // L
                m = e - k2 * L
                c0 = k2 * 32 + m
                plsc.store_scatter(outbuf, [rowb + c0], acc_lo)
                plsc.store_scatter(outbuf, [rowb + (c0 + L)], acc_hi)
            return 0
        lax.fori_loop(0, C // L, group_body, 0)

        pltpu.sync_copy(outbuf, out_hbm.at[pl.ds(row0 * D, C * D)])
        return 0
    lax.fori_loop(0, NCHUNK, chunk_body, 0)


@jax.jit
def _encode(x_flat, w_flat):
    mesh = plsc.VectorSubcoreMesh(
        core_axis_name="c", subcore_axis_name="s", num_cores=NC, num_subcores=NS)
    f = pl.kernel(
        _sc_body,
        out_type=jax.ShapeDtypeStruct((N * D,), jnp.float32),
        mesh=mesh,
        compiler_params=pltpu.CompilerParams(needs_layout_passes=False),
        scratch_types=[
            pltpu.VMEM((NT * R * D,), jnp.float32),
            pltpu.VMEM((W32 * TBL_ROWS,), jnp.float32),
            pltpu.VMEM((C * NT,), jnp.int32),
            pltpu.VMEM((C * D,), jnp.float32),
        ],
    )
    return f(w_flat, x_flat)


def kernel(x, W):
    x_flat = x.reshape(-1).astype(jnp.int32)
    w_flat = W.reshape(-1)
    out = _encode(x_flat, w_flat)
    return out.reshape(N, D)


# speedup vs baseline: 2.2961x; 2.2961x over previous
import jax
import jax.numpy as jnp
from jax import lax
from jax.experimental import pallas as pl
from jax.experimental.pallas import tpu as pltpu
from jax.experimental.pallas import tpu_sc as plsc

N = 320000
NT = 17
R = 8
D = 128
L = 16
NC, NS = 2, 16
NW = NC * NS
ROWS_PER_W = N // NW
C = 80
NCHUNK = ROWS_PER_W // C
GPC = C // L

NPAIR = 8
TS = D // 2
TBL_ROWS = NPAIR * 64 + R
NLOOK = NPAIR + 1


def _sc_body(w_hbm, x_hbm, out_hbm, wbuf, tbl, xbuf, outbuf):
    wid = lax.axis_index("s") * NC + lax.axis_index("c")
    base = wid * ROWS_PER_W

    pltpu.sync_copy(w_hbm, wbuf)

    iota = lax.iota(jnp.int32, L)

    def to_bf16_bits(v):
        w = lax.bitcast_convert_type(v, jnp.int32)
        rounded = w + 0x7FFF + ((w >> 16) & 1)
        return (rounded >> 16) & 0xFFFF

    def pack_row_to(vs, dst):
        for k in range(4):
            lo = to_bf16_bits(vs[2 * k])
            hi = to_bf16_bits(vs[2 * k + 1])
            word = lax.bitcast_convert_type(lo | (hi << 16), jnp.float32)
            tbl[pl.ds(dst + k * L, L)] = word

    def build_pair(p, _):
        def build_ab(ab, _):
            a = ab // R
            b = ab - a * R
            src_a = ((2 * p) * R + a) * D
            src_b = ((2 * p + 1) * R + b) * D
            vs = [wbuf[pl.ds(src_a + k * L, L)] + wbuf[pl.ds(src_b + k * L, L)]
                  for k in range(8)]
            pack_row_to(vs, (p * 64 + ab) * TS)
            return 0
        lax.fori_loop(0, 64, build_ab, 0)
        return 0
    lax.fori_loop(0, NPAIR, build_pair, 0)

    def build_single(r, _):
        src = (16 * R + r) * D
        vs = [wbuf[pl.ds(src + k * L, L)] for k in range(8)]
        pack_row_to(vs, (NPAIR * 64 + r) * TS)
        return 0
    lax.fori_loop(0, R, build_single, 0)

    xcol_base = iota * NT

    def chunk_body(j, _):
        row0 = base + j * C
        pltpu.sync_copy(x_hbm.at[pl.ds(row0 * NT, C * NT)], xbuf)

        def group_body(g, _):
            r0 = g * L
            xidx = xcol_base + r0 * NT
            cols = [plsc.load_gather(xbuf, [xidx + i]) for i in range(NT)]
            offs = []
            for p in range(NPAIR):
                offs.append((cols[2 * p] * R + cols[2 * p + 1] + p * 64) * TS)
            offs.append((cols[16] + NPAIR * 64) * TS)
            for r in range(L):
                osc = [off[r] for off in offs]
                ob = (r0 + r) * D
                for k in range(4):
                    acc = None
                    for t in range(NLOOK):
                        w = tbl[pl.ds(osc[t] + k * L, L)]
                        wb = plsc.bitcast(w, jnp.bfloat16)
                        acc = wb if acc is None else acc + wb
                    gi = plsc.bitcast(acc, jnp.int32)
                    lo = lax.bitcast_convert_type(gi << 16, jnp.float32)
                    hi = lax.bitcast_convert_type(gi & jnp.int32(-65536),
                                                  jnp.float32)
                    outbuf[pl.ds(ob + k * 32, L)] = lo
                    outbuf[pl.ds(ob + k * 32 + L, L)] = hi
            return 0
        lax.fori_loop(0, GPC, group_body, 0)

        pltpu.sync_copy(outbuf, out_hbm.at[pl.ds(row0 * D, C * D)])
        return 0
    lax.fori_loop(0, NCHUNK, chunk_body, 0)


@jax.jit
def _encode(x_flat, w_flat):
    mesh = plsc.VectorSubcoreMesh(
        core_axis_name="c", subcore_axis_name="s", num_cores=NC, num_subcores=NS)
    f = pl.kernel(
        _sc_body,
        out_type=jax.ShapeDtypeStruct((N * D,), jnp.float32),
        mesh=mesh,
        compiler_params=pltpu.CompilerParams(needs_layout_passes=False),
        scratch_types=[
            pltpu.VMEM((NT * R * D,), jnp.float32),
            pltpu.VMEM((TBL_ROWS * TS,), jnp.float32),
            pltpu.VMEM((C * NT,), jnp.int32),
            pltpu.VMEM((C * D,), jnp.float32),
        ],
    )
    return f(w_flat, x_flat)


def kernel(x, W):
    x_flat = x.reshape(-1).astype(jnp.int32)
    w_flat = W.reshape(-1)
    out = _encode(x_flat, w_flat)
    return out.reshape(N, D)
